# baseline (speedup 1.0000x reference)
import functools

import jax
import jax.numpy as jnp
from jax import lax
from jax.experimental import pallas as pl
from jax.experimental.pallas import tpu as pltpu
from jax.experimental.pallas import tpu_sc as plsc

D = 128
NSEQ = 4096
SEQ = 50
SEQP = 56
NC, NS = 2, 16
NW = NC * NS
SPW = NSEQ // NW
CSEQ = 4
CHUNK = CSEQ * SEQP
NCHUNK = SPW // CSEQ
BPW = SPW * SEQP
NBUF = 3

_mesh = plsc.VectorSubcoreMesh(core_axis_name="c", subcore_axis_name="s")


@functools.partial(
    pl.kernel,
    out_type=jax.ShapeDtypeStruct((NSEQ * SEQP, D), jnp.float32),
    mesh=_mesh,
    scratch_types=[
        pltpu.VMEM((BPW,), jnp.int32),
        pltpu.VMEM((NBUF, CHUNK, D), jnp.float32),
        [pltpu.SemaphoreType.DMA] * NBUF,
        [pltpu.SemaphoreType.DMA] * NBUF,
    ],
)
def _embed_gather(idx_hbm, table_hbm, out_hbm, idx_v, rows_v, gsems, ssems):
    wid = lax.axis_index("s") * NC + lax.axis_index("c")
    base = wid * BPW
    pltpu.sync_copy(idx_hbm.at[wid], idx_v)

    def drain_gather(b):
        pltpu.make_async_copy(
            table_hbm.at[pl.ds(0, CHUNK)], rows_v.at[b], gsems[b]
        ).wait()

    def drain_store(b):
        pltpu.make_async_copy(
            rows_v.at[b], out_hbm.at[pl.ds(base, CHUNK)], ssems[b]
        ).wait()

    def issue_gather(g, b):
        pltpu.async_copy(
            table_hbm.at[idx_v.at[pl.ds(g * CHUNK, CHUNK)]], rows_v.at[b], gsems[b]
        )

    def issue_store(g, b):
        pltpu.async_copy(
            rows_v.at[b], out_hbm.at[pl.ds(base + g * CHUNK, CHUNK)], ssems[b]
        )

    for b in range(NBUF - 1):
        issue_gather(b, b)

    MAIN = NCHUNK - NCHUNK % NBUF - NBUF

    @pl.loop(0, MAIN, step=NBUF)
    def _outer(go):
        for b in range(NBUF):
            g = go + b
            pb = (b - 1) % NBUF
            drain_gather(b)

            @pl.when(g > 0)
            def _():
                drain_store(pb)

            issue_gather(g + NBUF - 1, pb)
            issue_store(g, b)

    for g in range(MAIN, NCHUNK):
        b = g % NBUF
        pb = (b - 1) % NBUF
        drain_gather(b)
        if g > 0:
            drain_store(pb)
        if g + NBUF - 1 < NCHUNK:
            issue_gather(g + NBUF - 1, pb)
        issue_store(g, b)

    drain_store((NCHUNK - 1) % NBUF)


def kernel(x, table):
    idx = jnp.pad(x.astype(jnp.int32), ((0, 0), (0, SEQP - SEQ)))
    idx = idx.reshape(NW, BPW)
    out = _embed_gather(idx, table)
    return out.reshape(NSEQ, SEQP, D)[:, :SEQ, :]


# speedup vs baseline: 4.4226x; 4.4226x over previous
import functools

import jax
import jax.numpy as jnp
from jax import lax
from jax.experimental import pallas as pl
from jax.experimental.pallas import tpu as pltpu
from jax.experimental.pallas import tpu_sc as plsc

D = 128
B_TOTAL = 4096 * 50
NC, NS = 2, 16
NW = NC * NS
BPW = B_TOTAL // NW
CHUNK = 320
NCHUNK = BPW // CHUNK
NBUF = 3

_mesh = plsc.VectorSubcoreMesh(core_axis_name="c", subcore_axis_name="s")


@functools.partial(
    pl.kernel,
    out_type=jax.ShapeDtypeStruct((B_TOTAL, D), jnp.float32),
    mesh=_mesh,
    scratch_types=[
        pltpu.VMEM((BPW,), jnp.int32),
        pltpu.VMEM((NBUF, CHUNK, D), jnp.float32),
        [pltpu.SemaphoreType.DMA] * NBUF,
        [pltpu.SemaphoreType.DMA] * NBUF,
    ],
)
def _embed_gather(idx_hbm, table_hbm, out_hbm, idx_v, rows_v, gsems, ssems):
    wid = lax.axis_index("s") * NC + lax.axis_index("c")
    base = wid * BPW
    pltpu.sync_copy(idx_hbm.at[wid], idx_v)

    def drain_gather(b):
        pltpu.make_async_copy(
            table_hbm.at[pl.ds(0, CHUNK)], rows_v.at[b], gsems[b]
        ).wait()

    def drain_store(b):
        pltpu.make_async_copy(
            rows_v.at[b], out_hbm.at[pl.ds(base, CHUNK)], ssems[b]
        ).wait()

    def issue_gather(g, b):
        pltpu.async_copy(
            table_hbm.at[idx_v.at[pl.ds(g * CHUNK, CHUNK)]], rows_v.at[b], gsems[b]
        )

    def issue_store(g, b):
        pltpu.async_copy(
            rows_v.at[b], out_hbm.at[pl.ds(base + g * CHUNK, CHUNK)], ssems[b]
        )

    for b in range(NBUF - 1):
        issue_gather(b, b)

    MAIN = NCHUNK - NCHUNK % NBUF - NBUF

    @pl.loop(0, MAIN, step=NBUF)
    def _outer(go):
        for b in range(NBUF):
            g = go + b
            pb = (b - 1) % NBUF
            drain_gather(b)

            @pl.when(g > 0)
            def _():
                drain_store(pb)

            issue_gather(g + NBUF - 1, pb)
            issue_store(g, b)

    for g in range(MAIN, NCHUNK):
        b = g % NBUF
        pb = (b - 1) % NBUF
        drain_gather(b)
        if g > 0:
            drain_store(pb)
        if g + NBUF - 1 < NCHUNK:
            issue_gather(g + NBUF - 1, pb)
        issue_store(g, b)

    drain_store((NCHUNK - 1) % NBUF)


def kernel(x, table):
    idx = x.reshape(NW, BPW).astype(jnp.int32)
    out = _embed_gather(idx, table)
    return out.reshape(x.shape + (D,))
